# baseline (speedup 1.0000x reference)
import functools

import jax
import jax.numpy as jnp
from jax import lax
from jax.experimental import pallas as pl
from jax.experimental.pallas import tpu as pltpu
from jax.experimental.pallas import tpu_sc as plsc

B = 4096
C = 30
CPAD = 32
EDIM = 64
NT = 113
NW = 32
RPW = B // NW
CBI = CPAD
NCHUNK = RPW
L = 16
BLK = 8 * EDIM


def _sc_body(base_hbm, sub_hbm, ctx_hbm, tab_hbm, sco_hbm, prb_hbm,
             base_v, sub_v, ctx_v, emb_v, sco_v, prb_v, semg0, semg1):
    wid = lax.axis_index("s") * 2 + lax.axis_index("c")
    row0 = wid * RPW
    nidx = NCHUNK * CBI
    ibase = pl.multiple_of(wid * nidx, nidx)

    pltpu.sync_copy(base_hbm.at[pl.ds(ibase, nidx)], base_v)
    pltpu.sync_copy(sub_hbm.at[pl.ds(ibase, nidx)], sub_v)
    nctx = RPW * EDIM
    pltpu.sync_copy(
        ctx_hbm.at[pl.ds(pl.multiple_of(wid * nctx, nctx), nctx)], ctx_v)

    iota = lax.iota(jnp.int32, L)
    iota_blk = iota * BLK
    semg = (semg0, semg1)

    def issue_gathers(k, p):
        for g in range(2):
            vb = base_v[pl.ds(k * CBI + g * L, L)]
            for j in range(L):
                t = vb[j]
                s = g * L + j
                pltpu.async_copy(
                    tab_hbm.at[pl.ds(pl.multiple_of(t, 8), BLK)],
                    emb_v.at[p, pl.ds(s * BLK, BLK)], semg[p])

    def wait_gathers(p):
        pltpu.make_async_copy(tab_hbm.at[pl.ds(0, CBI * BLK)], emb_v.at[p],
                              semg[p]).wait()

    def compute_chunk(row, embp):
        rowb = jnp.broadcast_to(row * EDIM, (L,))
        addr0 = iota_blk + sub_v[pl.ds(row * CPAD, L)]
        addr1 = iota_blk + jnp.broadcast_to(L * BLK, (L,)) \
            + sub_v[pl.ds(row * CPAD + L, L)]

        def d_body(d, accs):
            a0, a1, b0, b1 = accs
            dv = jnp.broadcast_to(d, (L,))
            cb = plsc.load_gather(ctx_v, [rowb + dv])
            e0 = plsc.load_gather(embp, [addr0 + dv])
            e1 = plsc.load_gather(embp, [addr1 + dv])
            dv2 = dv + 1
            cb2 = plsc.load_gather(ctx_v, [rowb + dv2])
            e0b = plsc.load_gather(embp, [addr0 + dv2])
            e1b = plsc.load_gather(embp, [addr1 + dv2])
            return (a0 + cb * e0, a1 + cb * e1,
                    b0 + cb2 * e0b, b1 + cb2 * e1b)

        z = jnp.zeros((L,), jnp.float32)
        a0, a1, b0, b1 = lax.fori_loop(0, EDIM // 2,
                                       lambda i, acc: d_body(2 * i, acc),
                                       (z, z, z, z), unroll=8)
        s0 = a0 + b0
        s1 = a1 + b1

        m = jnp.maximum(jnp.max(s0), jnp.max(s1))
        mb = jnp.broadcast_to(m, (L,))
        e0 = jnp.exp(s0 - mb)
        e1 = jnp.where(iota < (C - L), jnp.exp(s1 - mb),
                       jnp.zeros((L,), jnp.float32))
        t = jnp.sum(e0) + jnp.sum(e1)
        invb = jnp.ones((L,), jnp.float32) / jnp.broadcast_to(t, (L,))
        base = row * CPAD
        sco_v[pl.ds(base, L)] = s0
        sco_v[pl.ds(base + L, L)] = s1
        prb_v[pl.ds(base, L)] = e0 * invb
        prb_v[pl.ds(base + L, L)] = e1 * invb

    issue_gathers(0, 0)
    issue_gathers(1, 1)

    def pair_body(kk, carry):
        for p in (0, 1):
            k = 2 * kk + p
            wait_gathers(p)
            compute_chunk(k, emb_v.at[p])

            @pl.when(k + 2 < NCHUNK)
            def _():
                issue_gathers(k + 2, p)
        return carry

    lax.fori_loop(0, NCHUNK // 2, pair_body, 0)

    obase = pl.multiple_of(row0 * CPAD, RPW * CPAD)
    pltpu.sync_copy(sco_v, sco_hbm.at[pl.ds(obase, RPW * CPAD)])
    pltpu.sync_copy(prb_v, prb_hbm.at[pl.ds(obase, RPW * CPAD)])


@functools.partial(
    pl.kernel,
    mesh=plsc.VectorSubcoreMesh(core_axis_name="c", subcore_axis_name="s"),
    compiler_params=pltpu.CompilerParams(
        needs_layout_passes=False, use_tc_tiling_on_sc=False),
    out_type=[
        jax.ShapeDtypeStruct((B * CPAD,), jnp.float32),
        jax.ShapeDtypeStruct((B * CPAD,), jnp.float32),
    ],
    scratch_types=[
        pltpu.VMEM((NCHUNK * CBI,), jnp.int32),
        pltpu.VMEM((NCHUNK * CBI,), jnp.int32),
        pltpu.VMEM((RPW * EDIM,), jnp.float32),
        pltpu.VMEM((2, CBI * BLK), jnp.float32),
        pltpu.VMEM((RPW * CPAD,), jnp.float32),
        pltpu.VMEM((RPW * CPAD,), jnp.float32),
        pltpu.SemaphoreType.DMA,
        pltpu.SemaphoreType.DMA,
    ],
)
def _sc_scores(base_hbm, sub_hbm, ctx_hbm, tab_hbm, sco_hbm, prb_hbm,
               *scratch):
    _sc_body(base_hbm, sub_hbm, ctx_hbm, tab_hbm, sco_hbm, prb_hbm, *scratch)


def _tc_body(ctx_ref, w_ref, b_ref, o_ref):
    y = jnp.dot(ctx_ref[...], w_ref[...],
                preferred_element_type=jnp.float32) + b_ref[...]
    o_ref[...] = jax.nn.sigmoid(y)


def _mentype(ctx, w, b2d):
    return pl.pallas_call(
        _tc_body,
        out_shape=jax.ShapeDtypeStruct((B, NT), jnp.float32),
    )(ctx, w, b2d)


def kernel(leftb, rightb, leftlens, rightlens, docb, wididxsb,
           entity_table, context_encoded, type_W, type_b):
    idx_pad = jnp.concatenate([wididxsb, wididxsb[:, C - 2:]], axis=1)
    base1d = ((idx_pad & ~7) * EDIM).reshape(-1)
    sub1d = ((idx_pad & 7) * EDIM).reshape(-1)
    tab1 = entity_table.reshape(-1)
    ctx1 = context_encoded.reshape(-1)
    sco_f, prb_f = _sc_scores(base1d, sub1d, ctx1, tab1)
    sco = sco_f.reshape(B, CPAD)[:, :C]
    prb = prb_f.reshape(B, CPAD)[:, :C]
    ment = _mentype(context_encoded, type_W, type_b.reshape(1, NT))
    return sco, prb, ment


# speedup vs baseline: 1.3450x; 1.3450x over previous
import functools

import jax
import jax.numpy as jnp
from jax import lax
from jax.experimental import pallas as pl
from jax.experimental.pallas import tpu as pltpu
from jax.experimental.pallas import tpu_sc as plsc

B = 4096
C = 30
CPAD = 32
EDIM = 64
NT = 113
NW = 32
RPW = B // NW
CBI = CPAD
NCHUNK = RPW
L = 16


def _sc_body(base_hbm, sub_hbm, ctx_hbm, tab_hbm, sco_hbm, prb_hbm,
             base_v, sub_v, ctx_v, emb_v, sco_v, prb_v, semg0, semg1):
    wid = lax.axis_index("s") * 2 + lax.axis_index("c")
    row0 = wid * RPW
    nidx = NCHUNK * CBI
    ibase = pl.multiple_of(wid * nidx, nidx)

    pltpu.sync_copy(base_hbm.at[pl.ds(ibase, nidx)], base_v)
    pltpu.sync_copy(sub_hbm.at[pl.ds(ibase, nidx)], sub_v)
    pltpu.sync_copy(
        ctx_hbm.at[pl.ds(pl.multiple_of(wid * (RPW // 2), RPW // 2),
                         RPW // 2)], ctx_v)

    iota = lax.iota(jnp.int32, L)
    iota8 = iota * 8
    semg = (semg0, semg1)

    def issue_gathers(k, p):
        for g in range(2):
            vb = base_v[pl.ds(k * CBI + g * L, L)]
            for j in range(L):
                t = vb[j]
                s = g * L + j
                pltpu.async_copy(
                    tab_hbm.at[pl.ds(pl.multiple_of(t, 8), 8)],
                    emb_v.at[p, pl.ds(s * 8, 8)], semg[p])

    def wait_gathers(p):
        pltpu.make_async_copy(tab_hbm.at[pl.ds(0, CBI * 8)], emb_v.at[p],
                              semg[p]).wait()

    def compute_chunk(row, embp):
        coff = (row & 1) * EDIM
        rowv = jnp.broadcast_to(row >> 1, (L,))
        coffv = jnp.broadcast_to(coff, (L,))
        rows0 = iota8 + sub_v[pl.ds(row * CPAD, L)]
        rows1 = iota8 + jnp.broadcast_to(L * 8, (L,)) \
            + sub_v[pl.ds(row * CPAD + L, L)]

        def d_body(d, accs):
            a0, a1, b0, b1 = accs
            dc = jnp.broadcast_to(d + coff, (L,))
            dv = dc - coffv
            cb = plsc.load_gather(ctx_v, [rowv, dc])
            e0 = plsc.load_gather(embp, [rows0, dv])
            e1 = plsc.load_gather(embp, [rows1, dv])
            dc2 = dc + 1
            dv2 = dv + 1
            cb2 = plsc.load_gather(ctx_v, [rowv, dc2])
            e0b = plsc.load_gather(embp, [rows0, dv2])
            e1b = plsc.load_gather(embp, [rows1, dv2])
            return (a0 + cb * e0, a1 + cb * e1,
                    b0 + cb2 * e0b, b1 + cb2 * e1b)

        z = jnp.zeros((L,), jnp.float32)
        a0, a1, b0, b1 = lax.fori_loop(0, EDIM // 2,
                                       lambda i, acc: d_body(2 * i, acc),
                                       (z, z, z, z), unroll=8)
        s0 = a0 + b0
        s1 = a1 + b1

        m = jnp.maximum(jnp.max(s0), jnp.max(s1))
        mb = jnp.broadcast_to(m, (L,))
        e0 = jnp.exp(s0 - mb)
        e1 = jnp.where(iota < (C - L), jnp.exp(s1 - mb),
                       jnp.zeros((L,), jnp.float32))
        t = jnp.sum(e0) + jnp.sum(e1)
        invb = jnp.ones((L,), jnp.float32) / jnp.broadcast_to(t, (L,))
        base = row * CPAD
        sco_v[pl.ds(base, L)] = s0
        sco_v[pl.ds(base + L, L)] = s1
        prb_v[pl.ds(base, L)] = e0 * invb
        prb_v[pl.ds(base + L, L)] = e1 * invb

    issue_gathers(0, 0)
    issue_gathers(1, 1)

    def pair_body(kk, carry):
        for p in (0, 1):
            k = 2 * kk + p
            wait_gathers(p)
            compute_chunk(k, emb_v.at[p])

            @pl.when(k + 2 < NCHUNK)
            def _():
                issue_gathers(k + 2, p)
        return carry

    lax.fori_loop(0, NCHUNK // 2, pair_body, 0)

    obase = pl.multiple_of(row0 * CPAD, RPW * CPAD)
    pltpu.sync_copy(sco_v, sco_hbm.at[pl.ds(obase, RPW * CPAD)])
    pltpu.sync_copy(prb_v, prb_hbm.at[pl.ds(obase, RPW * CPAD)])


@functools.partial(
    pl.kernel,
    mesh=plsc.VectorSubcoreMesh(core_axis_name="c", subcore_axis_name="s"),
    compiler_params=pltpu.CompilerParams(needs_layout_passes=False),
    out_type=[
        jax.ShapeDtypeStruct((B * CPAD,), jnp.float32),
        jax.ShapeDtypeStruct((B * CPAD,), jnp.float32),
    ],
    scratch_types=[
        pltpu.VMEM((NCHUNK * CBI,), jnp.int32),
        pltpu.VMEM((NCHUNK * CBI,), jnp.int32),
        pltpu.VMEM((RPW // 2, 2 * EDIM), jnp.float32),
        pltpu.VMEM((2, CBI * 8, EDIM), jnp.float32),
        pltpu.VMEM((RPW * CPAD,), jnp.float32),
        pltpu.VMEM((RPW * CPAD,), jnp.float32),
        pltpu.SemaphoreType.DMA,
        pltpu.SemaphoreType.DMA,
    ],
)
def _sc_scores(base_hbm, sub_hbm, ctx_hbm, tab_hbm, sco_hbm, prb_hbm,
               *scratch):
    _sc_body(base_hbm, sub_hbm, ctx_hbm, tab_hbm, sco_hbm, prb_hbm, *scratch)


def _tc_body(ctx_ref, w_ref, b_ref, o_ref):
    y = jnp.dot(ctx_ref[...], w_ref[...],
                preferred_element_type=jnp.float32) + b_ref[...]
    o_ref[...] = jax.nn.sigmoid(y)


def _mentype(ctx, w, b2d):
    return pl.pallas_call(
        _tc_body,
        out_shape=jax.ShapeDtypeStruct((B, NT), jnp.float32),
    )(ctx, w, b2d)


def kernel(leftb, rightb, leftlens, rightlens, docb, wididxsb,
           entity_table, context_encoded, type_W, type_b):
    idx_pad = jnp.concatenate([wididxsb, wididxsb[:, C - 2:]], axis=1)
    base1d = (idx_pad & ~7).reshape(-1)
    sub1d = (idx_pad & 7).reshape(-1)
    ctx2 = context_encoded.reshape(B // 2, 2 * EDIM)
    sco_f, prb_f = _sc_scores(base1d, sub1d, ctx2, entity_table)
    sco = sco_f.reshape(B, CPAD)[:, :C]
    prb = prb_f.reshape(B, CPAD)[:, :C]
    ment = _mentype(context_encoded, type_W, type_b.reshape(1, NT))
    return sco, prb, ment
